# baseline (speedup 1.0000x reference)
import functools

import jax
import jax.numpy as jnp
from jax import lax
from jax.experimental import pallas as pl
from jax.experimental.pallas import tpu as pltpu
from jax.experimental.pallas import tpu_sc as plsc

N_ENTITY = 10000
N_REL = 48
NUM_BASES = 8
D = 128
BATCH = 1024
HIST = 50

NC = 2
NS = 16
NW = NC * NS
CH = 128
NP = 10240
DR = NP // CH

_MESH = dict(core_axis_name="c", subcore_axis_name="s")


def _make_sc_gather(ncols, nchunks, nbuf):
    cpw = nchunks // NW
    stripe = NP // NS

    @functools.partial(
        pl.kernel,
        out_type=jax.ShapeDtypeStruct((nchunks * CH, ncols), jnp.float32),
        mesh=plsc.VectorSubcoreMesh(**_MESH),
        scratch_types=[
            pltpu.VMEM((cpw, 1, CH), jnp.int32),
            [pltpu.VMEM((CH, ncols), jnp.float32) for _ in range(nbuf)],
            [pltpu.SemaphoreType.DMA for _ in range(nbuf)],
            [pltpu.SemaphoreType.DMA for _ in range(nbuf)],
            pltpu.VMEM_SHARED((NP, ncols), jnp.float32),
        ],
    )
    def gather_kernel(table_hbm, idx_hbm, out_hbm, idx_v, rows, gsems, wsems,
                      tab_sh):
        cid = lax.axis_index("c")
        sid = lax.axis_index("s")
        wid = sid * NC + cid
        base = wid * cpw
        pltpu.sync_copy(table_hbm.at[pl.ds(sid * stripe, stripe)],
                        tab_sh.at[pl.ds(sid * stripe, stripe)])
        pltpu.sync_copy(idx_hbm.at[pl.ds(base, cpw)], idx_v)
        plsc.subcore_barrier()

        def gath(c, par):
            return pltpu.make_async_copy(tab_sh.at[idx_v.at[c, 0]], rows[par],
                                         gsems[par])

        def wrb(c, par):
            return pltpu.make_async_copy(
                rows[par], out_hbm.at[pl.ds((base + c) * CH, CH)], wsems[par])

        for par in range(nbuf):
            gath(par, par).start()

        def body(cc, carry):
            for par in range(nbuf):
                c = cc * nbuf + par
                gath(c, par).wait()
                w = wrb(c, par)
                w.start()
                w.wait()

                @pl.when(c + nbuf < cpw)
                def _():
                    gath(c + nbuf, par).start()

            return carry

        lax.fori_loop(0, cpw // nbuf, body, 0)

    return gather_kernel


def _make_sc_scatter(nchunks, nbuf):
    cpw = nchunks // NW
    rps = NP // NS

    @functools.partial(
        pl.kernel,
        out_type=(jax.ShapeDtypeStruct((2 * NP, D), jnp.float32),
                  jax.ShapeDtypeStruct((2 * DR, CH), jnp.float32)),
        mesh=plsc.VectorSubcoreMesh(**_MESH),
        scratch_types=[
            pltpu.VMEM((cpw, 1, CH), jnp.int32),
            [pltpu.VMEM((CH, D), jnp.float32) for _ in range(nbuf)],
            [pltpu.SemaphoreType.DMA for _ in range(nbuf)],
            pltpu.VMEM((DR, CH), jnp.float32),
            pltpu.VMEM((DR,), jnp.int32),
            pltpu.VMEM_SHARED((NP, D), jnp.float32),
            pltpu.VMEM_SHARED((DR, CH), jnp.float32),
        ],
        compiler_params=pltpu.CompilerParams(needs_layout_passes=False),
    )
    def scatter_kernel(m_hbm, dst_hbm, zeros_hbm, acc_out, deg_out,
                       idx_v, rows, msems, deg_v, iota_v, acc_sh, deg_sh):
        cid = lax.axis_index("c")
        sid = lax.axis_index("s")
        wid = sid * NC + cid
        base = wid * cpw
        pltpu.sync_copy(zeros_hbm.at[pl.ds(sid * rps, rps)],
                        acc_sh.at[pl.ds(sid * rps, rps)])

        @pl.when(sid == 0)
        def _():
            pltpu.sync_copy(zeros_hbm.at[pl.ds(0, DR)], deg_sh)

        pltpu.sync_copy(zeros_hbm.at[pl.ds(0, DR)], deg_v)
        pltpu.sync_copy(dst_hbm.at[pl.ds(base, cpw)], idx_v)

        def iloop(i, carry):
            iota_v[pl.ds(i * 16, 16)] = (
                lax.broadcasted_iota(jnp.int32, (16,), 0) + i * 16)
            return carry

        lax.fori_loop(0, DR // 16, iloop, 0)
        plsc.subcore_barrier()

        def mld(c, par):
            return pltpu.make_async_copy(
                m_hbm.at[pl.ds((base + c) * CH, CH)], rows[par], msems[par])

        for par in range(nbuf):
            mld(par, par).start()

        def body(cc, carry):
            for par in range(nbuf):
                c = cc * nbuf + par

                def dloop(k, carry2):
                    idx16 = idx_v[c, 0, pl.ds(k * 16, 16)]
                    counts, last = plsc.scan_count(idx16)
                    plsc.addupdate_scatter(
                        deg_v, [lax.shift_right_logical(idx16, 7),
                                lax.bitwise_and(idx16, CH - 1)],
                        counts.astype(jnp.float32), mask=last)
                    return carry2

                lax.fori_loop(0, CH // 16, dloop, 0)
                mld(c, par).wait()
                pltpu.sync_copy(rows[par], acc_sh.at[idx_v.at[c, 0]], add=True)

                @pl.when(c + nbuf < cpw)
                def _():
                    mld(c + nbuf, par).start()

            return carry

        lax.fori_loop(0, cpw // nbuf, body, 0)
        pltpu.sync_copy(deg_v, deg_sh.at[iota_v], add=True)
        plsc.subcore_barrier()
        pltpu.sync_copy(acc_sh.at[pl.ds(sid * rps, rps)],
                        acc_out.at[pl.ds(cid * NP + sid * rps, rps)])

        @pl.when(sid == 0)
        def _():
            pltpu.sync_copy(deg_sh, deg_out.at[pl.ds(cid * DR, DR)])

    return scatter_kernel


TB = 4096


def _tc_messages_body(xs_ref, et_ref, compb_ref, basis_ref, out_ref):
    x = xs_ref[...]
    et = et_ref[...]
    rel = lax.broadcasted_iota(jnp.int32, (TB, N_REL), 1)
    onehot = (et == rel).astype(jnp.float32)
    m = jnp.zeros((TB, D), jnp.float32)
    for b in range(NUM_BASES):
        cb = jnp.dot(onehot, compb_ref[b],
                     preferred_element_type=jnp.float32)
        y = jnp.dot(x, basis_ref[b], preferred_element_type=jnp.float32)
        m = m + cb * y
    out_ref[...] = m


TN = 1280


def _tc_kg_body(aggp_ref, degp_ref, ne_ref, rootW_ref, rootb_ref, out_ref):
    agg = aggp_ref[0] + aggp_ref[1] + aggp_ref[2] + aggp_ref[3]
    deg = degp_ref[0] + degp_ref[1] + degp_ref[2] + degp_ref[3]
    mean = agg / jnp.maximum(deg, 1.0)
    kg = mean + jnp.dot(ne_ref[...], rootW_ref[...],
                        preferred_element_type=jnp.float32) + rootb_ref[...]
    out_ref[...] = kg


TBB = 128
NAT = BATCH // TBB


def _tc_attn_head_body(H_ref, Wa_ref, a_ref, fc1W_ref, fc1b_ref, gamma_ref,
                       beta_ref, fc2W_ref, fc2b_ref, out_ref, h_scr):
    i = pl.program_id(0)

    @pl.when(i < NAT)
    def _():
        Hf = H_ref[...]
        H = Hf.reshape(TBB, HIST, D)
        t = jnp.tanh(jnp.dot(Hf, Wa_ref[...],
                             preferred_element_type=jnp.float32))
        e = jnp.dot(t, a_ref[...], preferred_element_type=jnp.float32)
        e3 = e.reshape(TBB, HIST, 1)
        emax = jnp.max(e3, axis=1, keepdims=True)
        ex = jnp.exp(e3 - emax)
        alpha = ex / jnp.sum(ex, axis=1, keepdims=True)
        profile = jnp.sum(alpha * H, axis=1)
        h_scr[pl.ds(i * TBB, TBB), :] = jnp.dot(
            profile, fc1W_ref[...],
            preferred_element_type=jnp.float32) + fc1b_ref[...]

    @pl.when(i == NAT)
    def _():
        h = h_scr[...]
        mu = jnp.mean(h, axis=0, keepdims=True)
        c = h - mu
        var = jnp.mean(c * c, axis=0, keepdims=True)
        hn = c / jnp.sqrt(var + 1e-5) * gamma_ref[...] + beta_ref[...]
        hr = jnp.maximum(hn, 0.0)
        out_ref[...] = jnp.dot(hr, fc2W_ref[...],
                               preferred_element_type=jnp.float32) + fc2b_ref[...]


def kernel(node_emb, basis, comp, root_W, root_b, attn_Wa, attn_a,
           fc1_W, fc1_b, bn_gamma, bn_beta, fc2_W, fc2_b,
           edge_index, edge_type, user_ids):
    E = edge_index.shape[1]
    echunks = -(-E // (CH * NW)) * NW
    E_pad = echunks * CH
    pad = E_pad - E

    src = edge_index[0].astype(jnp.int32)
    dst = edge_index[1].astype(jnp.int32)
    et = edge_type.astype(jnp.int32)
    src_p = jnp.concatenate([src, jnp.zeros((pad,), jnp.int32)])
    et_p = jnp.concatenate([et, jnp.full((pad,), N_REL, jnp.int32)])
    dst_p = jnp.concatenate(
        [dst, N_ENTITY + jnp.arange(pad, dtype=jnp.int32) % (NP - N_ENTITY)])
    et_col = et_p.reshape(E_pad, 1)
    ne_pad = jnp.concatenate(
        [node_emb, jnp.zeros((NP - N_ENTITY, D), jnp.float32)])

    NH = 2
    hchunks = echunks // NH
    EH = E_pad // NH
    comp_b = jnp.broadcast_to(comp.T[:, :, None], (NUM_BASES, N_REL, D))
    zeros_acc = jnp.zeros((NP, D), jnp.float32)
    gather_e = _make_sc_gather(D, hchunks, nbuf=2)
    scatter_e = _make_sc_scatter(hchunks, nbuf=2)
    ntiles = EH // TB

    xs_h = [gather_e(ne_pad, src_p[h * EH:(h + 1) * EH].reshape(hchunks, 1, CH))
            for h in range(NH)]
    m_h = [pl.pallas_call(
        _tc_messages_body,
        grid=(ntiles,),
        in_specs=[
            pl.BlockSpec((TB, D), lambda i: (i, 0)),
            pl.BlockSpec((TB, 1), lambda i: (i, 0)),
            pl.BlockSpec((NUM_BASES, N_REL, D), lambda i: (0, 0, 0)),
            pl.BlockSpec((NUM_BASES, D, D), lambda i: (0, 0, 0)),
        ],
        out_specs=pl.BlockSpec((TB, D), lambda i: (i, 0)),
        out_shape=jax.ShapeDtypeStruct((EH, D), jnp.float32),
    )(xs_h[h], et_col[h * EH:(h + 1) * EH], comp_b, basis)
        for h in range(NH)]
    parts = [scatter_e(m_h[h],
                       dst_p[h * EH:(h + 1) * EH].reshape(hchunks, 1, CH),
                       zeros_acc)
             for h in range(NH)]
    aggp = jnp.concatenate([p[0].reshape(2, NP, D) for p in parts])
    degp = jnp.concatenate([p[1].reshape(2, NP, 1) for p in parts])

    kg_pad = pl.pallas_call(
        _tc_kg_body,
        grid=(NP // TN,),
        in_specs=[
            pl.BlockSpec((2 * NH, TN, D), lambda i: (0, i, 0)),
            pl.BlockSpec((2 * NH, TN, 1), lambda i: (0, i, 0)),
            pl.BlockSpec((TN, D), lambda i: (i, 0)),
            pl.BlockSpec((D, D), lambda i: (0, 0)),
            pl.BlockSpec((1, D), lambda i: (0, 0)),
        ],
        out_specs=pl.BlockSpec((TN, D), lambda i: (i, 0)),
        out_shape=jax.ShapeDtypeStruct((NP, D), jnp.float32),
    )(aggp, degp, ne_pad, root_W, root_b.reshape(1, D))

    BU = BATCH * HIST
    uchunks = -(-BU // (CH * NW * 2)) * NW * 2
    BU_pad = uchunks * CH
    uid_p = jnp.concatenate([
        user_ids.reshape(-1).astype(jnp.int32),
        jnp.zeros((BU_pad - BU,), jnp.int32),
    ]).reshape(uchunks, 1, CH)
    H_full = _make_sc_gather(D, uchunks, nbuf=2)(kg_pad, uid_p)

    out = pl.pallas_call(
        _tc_attn_head_body,
        grid=(NAT + 1,),
        in_specs=[
            pl.BlockSpec((TBB * HIST, D),
                         lambda i: (jnp.minimum(i, NAT - 1), 0)),
            pl.BlockSpec((D, D), lambda i: (0, 0)),
            pl.BlockSpec((D, 1), lambda i: (0, 0)),
            pl.BlockSpec((D, D), lambda i: (0, 0)),
            pl.BlockSpec((1, D), lambda i: (0, 0)),
            pl.BlockSpec((1, D), lambda i: (0, 0)),
            pl.BlockSpec((1, D), lambda i: (0, 0)),
            pl.BlockSpec((D, D), lambda i: (0, 0)),
            pl.BlockSpec((1, D), lambda i: (0, 0)),
        ],
        out_specs=pl.BlockSpec((BATCH, D), lambda i: (0, 0)),
        out_shape=jax.ShapeDtypeStruct((BATCH, D), jnp.float32),
        scratch_shapes=[pltpu.VMEM((BATCH, D), jnp.float32)],
    )(H_full, attn_Wa, attn_a.reshape(D, 1), fc1_W, fc1_b.reshape(1, D),
      bn_gamma.reshape(1, D), bn_beta.reshape(1, D), fc2_W,
      fc2_b.reshape(1, D))
    return out


# speedup vs baseline: 1.0115x; 1.0115x over previous
import functools

import jax
import jax.numpy as jnp
from jax import lax
from jax.experimental import pallas as pl
from jax.experimental.pallas import tpu as pltpu
from jax.experimental.pallas import tpu_sc as plsc

N_ENTITY = 10000
N_REL = 48
NUM_BASES = 8
D = 128
BATCH = 1024
HIST = 50

NC = 2
NS = 16
NW = NC * NS
CH = 128
NP = 10240
DR = NP // CH

_MESH = dict(core_axis_name="c", subcore_axis_name="s")


def _make_sc_gather(ncols, nchunks, nbuf):
    cpw = nchunks // NW
    stripe = NP // NS

    @functools.partial(
        pl.kernel,
        out_type=jax.ShapeDtypeStruct((nchunks * CH, ncols), jnp.float32),
        mesh=plsc.VectorSubcoreMesh(**_MESH),
        scratch_types=[
            pltpu.VMEM((cpw, 1, CH), jnp.int32),
            [pltpu.VMEM((CH, ncols), jnp.float32) for _ in range(nbuf)],
            [pltpu.SemaphoreType.DMA for _ in range(nbuf)],
            [pltpu.SemaphoreType.DMA for _ in range(nbuf)],
            pltpu.VMEM_SHARED((NP, ncols), jnp.float32),
        ],
    )
    def gather_kernel(table_hbm, idx_hbm, out_hbm, idx_v, rows, gsems, wsems,
                      tab_sh):
        cid = lax.axis_index("c")
        sid = lax.axis_index("s")
        wid = sid * NC + cid
        base = wid * cpw
        pltpu.sync_copy(table_hbm.at[pl.ds(sid * stripe, stripe)],
                        tab_sh.at[pl.ds(sid * stripe, stripe)])
        pltpu.sync_copy(idx_hbm.at[pl.ds(base, cpw)], idx_v)
        plsc.subcore_barrier()

        def gath(c, par):
            return pltpu.make_async_copy(tab_sh.at[idx_v.at[c, 0]], rows[par],
                                         gsems[par])

        def wrb(c, par):
            return pltpu.make_async_copy(
                rows[par], out_hbm.at[pl.ds((base + c) * CH, CH)], wsems[par])

        for par in range(nbuf):
            gath(par, par).start()

        def body(cc, carry):
            for par in range(nbuf):
                c = cc * nbuf + par
                gath(c, par).wait()
                w = wrb(c, par)
                w.start()
                w.wait()

                @pl.when(c + nbuf < cpw)
                def _():
                    gath(c + nbuf, par).start()

            return carry

        lax.fori_loop(0, cpw // nbuf, body, 0)

    return gather_kernel


def _make_sc_scatter(nchunks, nbuf):
    cpw = nchunks // NW
    rps = NP // NS

    @functools.partial(
        pl.kernel,
        out_type=(jax.ShapeDtypeStruct((2 * NP, D), jnp.float32),
                  jax.ShapeDtypeStruct((2 * DR, CH), jnp.float32)),
        mesh=plsc.VectorSubcoreMesh(**_MESH),
        scratch_types=[
            pltpu.VMEM((cpw, 1, CH), jnp.int32),
            [pltpu.VMEM((CH, D), jnp.float32) for _ in range(nbuf)],
            [pltpu.SemaphoreType.DMA for _ in range(nbuf)],
            pltpu.VMEM((DR, CH), jnp.float32),
            pltpu.VMEM((DR,), jnp.int32),
            pltpu.VMEM_SHARED((NP, D), jnp.float32),
            pltpu.VMEM_SHARED((DR, CH), jnp.float32),
        ],
        compiler_params=pltpu.CompilerParams(needs_layout_passes=False),
    )
    def scatter_kernel(m_hbm, dst_hbm, zeros_hbm, acc_out, deg_out,
                       idx_v, rows, msems, deg_v, iota_v, acc_sh, deg_sh):
        cid = lax.axis_index("c")
        sid = lax.axis_index("s")
        wid = sid * NC + cid
        base = wid * cpw
        pltpu.sync_copy(zeros_hbm.at[pl.ds(sid * rps, rps)],
                        acc_sh.at[pl.ds(sid * rps, rps)])

        @pl.when(sid == 0)
        def _():
            pltpu.sync_copy(zeros_hbm.at[pl.ds(0, DR)], deg_sh)

        pltpu.sync_copy(zeros_hbm.at[pl.ds(0, DR)], deg_v)
        pltpu.sync_copy(dst_hbm.at[pl.ds(base, cpw)], idx_v)

        def iloop(i, carry):
            iota_v[pl.ds(i * 16, 16)] = (
                lax.broadcasted_iota(jnp.int32, (16,), 0) + i * 16)
            return carry

        lax.fori_loop(0, DR // 16, iloop, 0)
        plsc.subcore_barrier()

        def mld(c, par):
            return pltpu.make_async_copy(
                m_hbm.at[pl.ds((base + c) * CH, CH)], rows[par], msems[par])

        for par in range(nbuf):
            mld(par, par).start()

        def body(cc, carry):
            for par in range(nbuf):
                c = cc * nbuf + par

                def dloop(k, carry2):
                    idx16 = idx_v[c, 0, pl.ds(k * 16, 16)]
                    counts, last = plsc.scan_count(idx16)
                    plsc.addupdate_scatter(
                        deg_v, [lax.shift_right_logical(idx16, 7),
                                lax.bitwise_and(idx16, CH - 1)],
                        counts.astype(jnp.float32), mask=last)
                    return carry2

                lax.fori_loop(0, CH // 16, dloop, 0)
                mld(c, par).wait()
                pltpu.sync_copy(rows[par], acc_sh.at[idx_v.at[c, 0]], add=True)

                @pl.when(c + nbuf < cpw)
                def _():
                    mld(c + nbuf, par).start()

            return carry

        lax.fori_loop(0, cpw // nbuf, body, 0)
        pltpu.sync_copy(deg_v, deg_sh.at[iota_v], add=True)
        plsc.subcore_barrier()
        pltpu.sync_copy(acc_sh.at[pl.ds(sid * rps, rps)],
                        acc_out.at[pl.ds(cid * NP + sid * rps, rps)])

        @pl.when(sid == 0)
        def _():
            pltpu.sync_copy(deg_sh, deg_out.at[pl.ds(cid * DR, DR)])

    return scatter_kernel


TB = 4096


def _tc_messages_body(xs_ref, et_ref, compb_ref, basis_ref, out_ref):
    x = xs_ref[...]
    et = et_ref[...]
    rel = lax.broadcasted_iota(jnp.int32, (TB, N_REL), 1)
    onehot = (et == rel).astype(jnp.float32)
    m = jnp.zeros((TB, D), jnp.float32)
    for b in range(NUM_BASES):
        cb = jnp.dot(onehot, compb_ref[b],
                     preferred_element_type=jnp.float32)
        y = jnp.dot(x, basis_ref[b], preferred_element_type=jnp.float32)
        m = m + cb * y
    out_ref[...] = m


TN = 1280


def _tc_kg_body(aggp_ref, degp_ref, ne_ref, rootW_ref, rootb_ref, out_ref):
    nparts = aggp_ref.shape[0]
    agg = aggp_ref[0]
    deg = degp_ref[0]
    for k in range(1, nparts):
        agg = agg + aggp_ref[k]
        deg = deg + degp_ref[k]
    mean = agg / jnp.maximum(deg, 1.0)
    kg = mean + jnp.dot(ne_ref[...], rootW_ref[...],
                        preferred_element_type=jnp.float32) + rootb_ref[...]
    out_ref[...] = kg


TBB = 128
NAT = BATCH // TBB


def _tc_attn_head_body(H_ref, Wa_ref, a_ref, fc1W_ref, fc1b_ref, gamma_ref,
                       beta_ref, fc2W_ref, fc2b_ref, out_ref, h_scr):
    i = pl.program_id(0)

    @pl.when(i < NAT)
    def _():
        Hf = H_ref[...]
        H = Hf.reshape(TBB, HIST, D)
        t = jnp.tanh(jnp.dot(Hf, Wa_ref[...],
                             preferred_element_type=jnp.float32))
        e = jnp.dot(t, a_ref[...], preferred_element_type=jnp.float32)
        e3 = e.reshape(TBB, HIST, 1)
        emax = jnp.max(e3, axis=1, keepdims=True)
        ex = jnp.exp(e3 - emax)
        alpha = ex / jnp.sum(ex, axis=1, keepdims=True)
        profile = jnp.sum(alpha * H, axis=1)
        h_scr[pl.ds(i * TBB, TBB), :] = jnp.dot(
            profile, fc1W_ref[...],
            preferred_element_type=jnp.float32) + fc1b_ref[...]

    @pl.when(i == NAT)
    def _():
        h = h_scr[...]
        mu = jnp.mean(h, axis=0, keepdims=True)
        c = h - mu
        var = jnp.mean(c * c, axis=0, keepdims=True)
        hn = c / jnp.sqrt(var + 1e-5) * gamma_ref[...] + beta_ref[...]
        hr = jnp.maximum(hn, 0.0)
        out_ref[...] = jnp.dot(hr, fc2W_ref[...],
                               preferred_element_type=jnp.float32) + fc2b_ref[...]


def kernel(node_emb, basis, comp, root_W, root_b, attn_Wa, attn_a,
           fc1_W, fc1_b, bn_gamma, bn_beta, fc2_W, fc2_b,
           edge_index, edge_type, user_ids):
    E = edge_index.shape[1]
    echunks = -(-E // (CH * NW)) * NW
    E_pad = echunks * CH
    pad = E_pad - E

    src = edge_index[0].astype(jnp.int32)
    dst = edge_index[1].astype(jnp.int32)
    et = edge_type.astype(jnp.int32)
    src_p = jnp.concatenate([src, jnp.zeros((pad,), jnp.int32)])
    et_p = jnp.concatenate([et, jnp.full((pad,), N_REL, jnp.int32)])
    dst_p = jnp.concatenate(
        [dst, N_ENTITY + jnp.arange(pad, dtype=jnp.int32) % (NP - N_ENTITY)])
    et_col = et_p.reshape(E_pad, 1)
    ne_pad = jnp.concatenate(
        [node_emb, jnp.zeros((NP - N_ENTITY, D), jnp.float32)])

    NH = 1
    hchunks = echunks // NH
    EH = E_pad // NH
    comp_b = jnp.broadcast_to(comp.T[:, :, None], (NUM_BASES, N_REL, D))
    zeros_acc = jnp.zeros((NP, D), jnp.float32)
    gather_e = _make_sc_gather(D, hchunks, nbuf=2)
    scatter_e = _make_sc_scatter(hchunks, nbuf=2)
    ntiles = EH // TB

    xs_h = [gather_e(ne_pad, src_p[h * EH:(h + 1) * EH].reshape(hchunks, 1, CH))
            for h in range(NH)]
    m_h = [pl.pallas_call(
        _tc_messages_body,
        grid=(ntiles,),
        in_specs=[
            pl.BlockSpec((TB, D), lambda i: (i, 0)),
            pl.BlockSpec((TB, 1), lambda i: (i, 0)),
            pl.BlockSpec((NUM_BASES, N_REL, D), lambda i: (0, 0, 0)),
            pl.BlockSpec((NUM_BASES, D, D), lambda i: (0, 0, 0)),
        ],
        out_specs=pl.BlockSpec((TB, D), lambda i: (i, 0)),
        out_shape=jax.ShapeDtypeStruct((EH, D), jnp.float32),
    )(xs_h[h], et_col[h * EH:(h + 1) * EH], comp_b, basis)
        for h in range(NH)]
    parts = [scatter_e(m_h[h],
                       dst_p[h * EH:(h + 1) * EH].reshape(hchunks, 1, CH),
                       zeros_acc)
             for h in range(NH)]
    aggp = jnp.concatenate([p[0].reshape(2, NP, D) for p in parts])
    degp = jnp.concatenate([p[1].reshape(2, NP, 1) for p in parts])

    kg_pad = pl.pallas_call(
        _tc_kg_body,
        grid=(NP // TN,),
        in_specs=[
            pl.BlockSpec((2 * NH, TN, D), lambda i: (0, i, 0)),
            pl.BlockSpec((2 * NH, TN, 1), lambda i: (0, i, 0)),
            pl.BlockSpec((TN, D), lambda i: (i, 0)),
            pl.BlockSpec((D, D), lambda i: (0, 0)),
            pl.BlockSpec((1, D), lambda i: (0, 0)),
        ],
        out_specs=pl.BlockSpec((TN, D), lambda i: (i, 0)),
        out_shape=jax.ShapeDtypeStruct((NP, D), jnp.float32),
    )(aggp, degp, ne_pad, root_W, root_b.reshape(1, D))

    BU = BATCH * HIST
    uchunks = -(-BU // (CH * NW * 2)) * NW * 2
    BU_pad = uchunks * CH
    uid_p = jnp.concatenate([
        user_ids.reshape(-1).astype(jnp.int32),
        jnp.zeros((BU_pad - BU,), jnp.int32),
    ]).reshape(uchunks, 1, CH)
    H_full = _make_sc_gather(D, uchunks, nbuf=2)(kg_pad, uid_p)

    out = pl.pallas_call(
        _tc_attn_head_body,
        grid=(NAT + 1,),
        in_specs=[
            pl.BlockSpec((TBB * HIST, D),
                         lambda i: (jnp.minimum(i, NAT - 1), 0)),
            pl.BlockSpec((D, D), lambda i: (0, 0)),
            pl.BlockSpec((D, 1), lambda i: (0, 0)),
            pl.BlockSpec((D, D), lambda i: (0, 0)),
            pl.BlockSpec((1, D), lambda i: (0, 0)),
            pl.BlockSpec((1, D), lambda i: (0, 0)),
            pl.BlockSpec((1, D), lambda i: (0, 0)),
            pl.BlockSpec((D, D), lambda i: (0, 0)),
            pl.BlockSpec((1, D), lambda i: (0, 0)),
        ],
        out_specs=pl.BlockSpec((BATCH, D), lambda i: (0, 0)),
        out_shape=jax.ShapeDtypeStruct((BATCH, D), jnp.float32),
        scratch_shapes=[pltpu.VMEM((BATCH, D), jnp.float32)],
    )(H_full, attn_Wa, attn_a.reshape(D, 1), fc1_W, fc1_b.reshape(1, D),
      bn_gamma.reshape(1, D), bn_beta.reshape(1, D), fc2_W,
      fc2_b.reshape(1, D))
    return out
